# baseline (speedup 1.0000x reference)
import jax
import jax.numpy as jnp
from jax import lax
from jax.experimental import pallas as pl
from jax.experimental.pallas import tpu as pltpu

BN_EPS = 1e-5
SLOPE = 0.01


def _interp_matrix(n_in):
    n_out = 2 * n_in
    s = jnp.arange(n_out, dtype=jnp.float32) * (n_in - 1) / (n_out - 1)
    i = jnp.arange(n_in, dtype=jnp.float32)
    return jnp.maximum(0.0, 1.0 - jnp.abs(s[:, None] - i[None, :]))


def _conv1x1_kernel(x_ref, w_ref, b_ref, o_ref):
    x = x_ref[0].astype(jnp.bfloat16)
    y = jnp.dot(w_ref[...], x, preferred_element_type=jnp.float32)
    o_ref[0] = y + b_ref[...]


def _xinterp_kernel(x_ref, uxt_ref, o_ref):
    t = jnp.dot(x_ref[0], uxt_ref[...],
                preferred_element_type=jnp.float32)
    o_ref[0] = t.T


def _yinterp_kernel(x_ref, uyt_ref, o_ref):
    w2, c2, hh = x_ref.shape[1], x_ref.shape[2], x_ref.shape[3]
    xin = x_ref[0].reshape(w2 * c2, hh)
    ym = jnp.dot(xin, uyt_ref[...],
                 preferred_element_type=jnp.float32)
    o_ref[0] = ym.T.astype(jnp.bfloat16)


def _make_conv3x3_kernel(n_in, h, w, forms, nchw_out):
    n = h * w
    guard = w

    def body(*refs):
        x_refs = refs[:n_in]
        w_refs = refs[n_in:2 * n_in]
        scale_ref, shift_ref, o_ref = refs[2 * n_in:2 * n_in + 3]
        scratch_refs = refs[2 * n_in + 3:]
        c_out = w_refs[0].shape[-1]

        for i, (x_ref, form) in enumerate(zip(x_refs, forms)):
            xin = x_ref[0]
            if xin.dtype != jnp.bfloat16:
                xin = xin.astype(jnp.bfloat16)
            if form == "cm":
                rows = xin.T
            elif form == "rows4":
                rows = xin.reshape(n, xin.shape[-1])
            else:
                rows = xin
            c_in = rows.shape[1]
            col = lax.broadcasted_iota(jnp.int32, (n, c_in), 0) % w
            zeros_g = jnp.zeros((guard + 1, c_in), jnp.bfloat16)
            m0 = jnp.where(col == w - 1, jnp.bfloat16(0), rows)
            m2 = jnp.where(col == 0, jnp.bfloat16(0), rows)
            for dx, val in ((0, m0), (1, rows), (2, m2)):
                s_ref = scratch_refs[3 * i + dx]
                off = guard + 1 - dx
                s_ref[:off, :] = zeros_g[:off, :]
                s_ref[off:off + n, :] = val
                s_ref[off + n:, :] = zeros_g[:2 * guard - off, :]

        acc = jnp.zeros((n, c_out), jnp.float32)
        for i, w_ref in enumerate(w_refs):
            for dx in range(3):
                s_ref = scratch_refs[3 * i + dx]
                for dy in range(3):
                    base = dy * w
                    acc = acc + jnp.dot(s_ref[base:base + n, :],
                                        w_ref[dy, dx],
                                        preferred_element_type=jnp.float32)

        y = acc * scale_ref[...] + shift_ref[...]
        y = jnp.where(y > 0, y, SLOPE * y)
        if nchw_out:
            o_ref[0] = y.T.astype(o_ref.dtype)
        else:
            o_ref[0] = y.astype(o_ref.dtype)

    return body


def _fold_bn(conv_bias, gamma, beta, mean, var):
    scale = gamma / jnp.sqrt(var + BN_EPS)
    shift = (conv_bias - mean) * scale + beta
    return (scale.astype(jnp.float32).reshape(1, -1),
            shift.astype(jnp.float32).reshape(1, -1))


def kernel(x1, x2, w_1x1, b_1x1, w_conv_a, b_conv_a,
           bn_a_gamma, bn_a_beta, bn_a_mean, bn_a_var,
           w_conv_b, b_conv_b, bn_b_gamma, bn_b_beta, bn_b_mean, bn_b_var):
    b, c1, h, w = x1.shape
    c2 = w_1x1.shape[0]
    c_out = w_conv_a.shape[0]
    h2, w2 = 2 * h, 2 * w
    n = h2 * w2
    guard = w2
    par = pltpu.CompilerParams(dimension_semantics=("parallel",))

    w1 = w_1x1[:, :, 0, 0].astype(jnp.bfloat16)
    y1 = pl.pallas_call(
        _conv1x1_kernel,
        out_shape=jax.ShapeDtypeStruct((b, c2, h * w), jnp.float32),
        grid=(b,),
        in_specs=[
            pl.BlockSpec((1, c1, h * w), lambda i: (i, 0, 0)),
            pl.BlockSpec((c2, c1), lambda i: (0, 0)),
            pl.BlockSpec((c2, 1), lambda i: (0, 0)),
        ],
        out_specs=pl.BlockSpec((1, c2, h * w), lambda i: (i, 0, 0)),
        compiler_params=par,
    )(x1.reshape(b, c1, h * w), w1, b_1x1.reshape(c2, 1).astype(jnp.float32))

    uxt = _interp_matrix(w).T
    uyt = _interp_matrix(h).T
    tx = pl.pallas_call(
        _xinterp_kernel,
        out_shape=jax.ShapeDtypeStruct((b, w2, c2 * h), jnp.float32),
        grid=(b,),
        in_specs=[
            pl.BlockSpec((1, c2 * h, w), lambda i: (i, 0, 0)),
            pl.BlockSpec((w, w2), lambda i: (0, 0)),
        ],
        out_specs=pl.BlockSpec((1, w2, c2 * h), lambda i: (i, 0, 0)),
        compiler_params=par,
    )(y1.reshape(b, c2 * h, w), uxt)

    y_up = pl.pallas_call(
        _yinterp_kernel,
        out_shape=jax.ShapeDtypeStruct((b, h2, w2 * c2), jnp.bfloat16),
        grid=(b,),
        in_specs=[
            pl.BlockSpec((1, w2, c2, h), lambda i: (i, 0, 0, 0)),
            pl.BlockSpec((h, h2), lambda i: (0, 0)),
        ],
        out_specs=pl.BlockSpec((1, h2, w2 * c2), lambda i: (i, 0, 0)),
        compiler_params=par,
    )(tx.reshape(b, w2, c2, h), uyt)

    w_a = jnp.transpose(w_conv_a, (2, 3, 1, 0)).astype(jnp.bfloat16)
    scale_a, shift_a = _fold_bn(b_conv_a, bn_a_gamma, bn_a_beta,
                                bn_a_mean, bn_a_var)
    za = pl.pallas_call(
        _make_conv3x3_kernel(2, h2, w2, forms=("cm", "rows4"),
                             nchw_out=False),
        out_shape=jax.ShapeDtypeStruct((b, n, c_out), jnp.bfloat16),
        grid=(b,),
        in_specs=[
            pl.BlockSpec((1, c2, n), lambda i: (i, 0, 0)),
            pl.BlockSpec((1, h2, w2, c2), lambda i: (i, 0, 0, 0)),
            pl.BlockSpec((3, 3, c2, c_out), lambda i: (0, 0, 0, 0)),
            pl.BlockSpec((3, 3, c2, c_out), lambda i: (0, 0, 0, 0)),
            pl.BlockSpec((1, c_out), lambda i: (0, 0)),
            pl.BlockSpec((1, c_out), lambda i: (0, 0)),
        ],
        out_specs=pl.BlockSpec((1, n, c_out), lambda i: (i, 0, 0)),
        compiler_params=par,
        scratch_shapes=[pltpu.VMEM((n + 2 * guard, c2), jnp.bfloat16)
                        for _ in range(6)],
    )(x2.reshape(b, c2, n), y_up.reshape(b, h2, w2, c2),
      w_a[:, :, :c2, :], w_a[:, :, c2:, :], scale_a, shift_a)

    w_b = jnp.transpose(w_conv_b, (2, 3, 1, 0)).astype(jnp.bfloat16)
    scale_b, shift_b = _fold_bn(b_conv_b, bn_b_gamma, bn_b_beta,
                                bn_b_mean, bn_b_var)
    zb = pl.pallas_call(
        _make_conv3x3_kernel(1, h2, w2, forms=("rows3",), nchw_out=True),
        out_shape=jax.ShapeDtypeStruct((b, c_out, n), jnp.float32),
        grid=(b,),
        in_specs=[
            pl.BlockSpec((1, n, c_out), lambda i: (i, 0, 0)),
            pl.BlockSpec((3, 3, c_out, c_out), lambda i: (0, 0, 0, 0)),
            pl.BlockSpec((1, c_out), lambda i: (0, 0)),
            pl.BlockSpec((1, c_out), lambda i: (0, 0)),
        ],
        out_specs=pl.BlockSpec((1, c_out, n), lambda i: (i, 0, 0)),
        compiler_params=par,
        scratch_shapes=[pltpu.VMEM((n + 2 * guard, c_out), jnp.bfloat16)
                        for _ in range(3)],
    )(za, w_b, scale_b, shift_b)

    return zb.reshape(b, c_out, h2, w2)


# speedup vs baseline: 1.2434x; 1.2434x over previous
import jax
import jax.numpy as jnp
from jax import lax
from jax.experimental import pallas as pl
from jax.experimental.pallas import tpu as pltpu

BN_EPS = 1e-5
SLOPE = 0.01


def _interp_matrix(n_in):
    n_out = 2 * n_in
    s = jnp.arange(n_out, dtype=jnp.float32) * (n_in - 1) / (n_out - 1)
    i = jnp.arange(n_in, dtype=jnp.float32)
    return jnp.maximum(0.0, 1.0 - jnp.abs(s[:, None] - i[None, :]))


def _conv1x1_kernel(x_ref, w_ref, b_ref, o_ref):
    x = x_ref[0].astype(jnp.bfloat16)
    y = jnp.dot(w_ref[...], x, preferred_element_type=jnp.float32)
    o_ref[0] = y + b_ref[...]


def _make_upsample_kernel(c2, h, w):
    h2, w2 = 2 * h, 2 * w

    def body(x_ref, uyb_ref, uxt_ref, o_ref):
        t = jnp.dot(x_ref[0], uxt_ref[...],
                    preferred_element_type=jnp.float32)
        t = t.reshape(c2, h, w2)
        o = lax.dot_general(uyb_ref[...], t, (((2,), (1,)), ((0,), (0,))),
                            preferred_element_type=jnp.float32)
        o_ref[0] = o.astype(jnp.bfloat16)

    return body


def _make_conv3x3_kernel(n_in, h, w, forms, nchw_out):
    n = h * w
    guard = w + 1

    def body(*refs):
        x_refs = refs[:n_in]
        w_refs = refs[n_in:2 * n_in]
        scale_ref, shift_ref, o_ref = refs[2 * n_in:2 * n_in + 3]
        scratch_refs = refs[2 * n_in + 3:]
        c_out = w_refs[0].shape[-1]

        for x_ref, form, s_ref in zip(x_refs, forms, scratch_refs):
            xin = x_ref[0]
            if xin.dtype != jnp.bfloat16:
                xin = xin.astype(jnp.bfloat16)
            if form == "cm":
                rows = xin.T
            else:
                rows = xin
            s_ref[:guard, :] = jnp.zeros((guard, rows.shape[1]), jnp.bfloat16)
            s_ref[guard:guard + n, :] = rows
            s_ref[guard + n:, :] = jnp.zeros((guard, rows.shape[1]),
                                             jnp.bfloat16)

        col = lax.broadcasted_iota(jnp.int32, (n, c_out), 0) % w
        acc = jnp.zeros((n, c_out), jnp.float32)
        for dx in range(3):
            part = jnp.zeros((n, c_out), jnp.float32)
            for s_ref, w_ref in zip(scratch_refs, w_refs):
                for dy in range(3):
                    base = guard + (dy - 1) * w + (dx - 1)
                    part = part + jnp.dot(s_ref[base:base + n, :],
                                          w_ref[dy, dx],
                                          preferred_element_type=jnp.float32)
            if dx == 0:
                part = jnp.where(col >= 1, part, 0.0)
            elif dx == 2:
                part = jnp.where(col < w - 1, part, 0.0)
            acc = acc + part

        y = acc * scale_ref[...] + shift_ref[...]
        y = jnp.where(y > 0, y, SLOPE * y)
        if nchw_out:
            o_ref[0] = y.T.astype(o_ref.dtype)
        else:
            o_ref[0] = y.astype(o_ref.dtype)

    return body


def _fold_bn(conv_bias, gamma, beta, mean, var):
    scale = gamma / jnp.sqrt(var + BN_EPS)
    shift = (conv_bias - mean) * scale + beta
    return (scale.astype(jnp.float32).reshape(1, -1),
            shift.astype(jnp.float32).reshape(1, -1))


def kernel(x1, x2, w_1x1, b_1x1, w_conv_a, b_conv_a,
           bn_a_gamma, bn_a_beta, bn_a_mean, bn_a_var,
           w_conv_b, b_conv_b, bn_b_gamma, bn_b_beta, bn_b_mean, bn_b_var):
    b, c1, h, w = x1.shape
    c2 = w_1x1.shape[0]
    c_out = w_conv_a.shape[0]
    h2, w2 = 2 * h, 2 * w
    n = h2 * w2
    guard = w2 + 1
    par = pltpu.CompilerParams(dimension_semantics=("parallel",))

    w1 = w_1x1[:, :, 0, 0].astype(jnp.bfloat16)
    y1 = pl.pallas_call(
        _conv1x1_kernel,
        out_shape=jax.ShapeDtypeStruct((b, c2, h * w), jnp.float32),
        grid=(b,),
        in_specs=[
            pl.BlockSpec((1, c1, h * w), lambda i: (i, 0, 0)),
            pl.BlockSpec((c2, c1), lambda i: (0, 0)),
            pl.BlockSpec((c2, 1), lambda i: (0, 0)),
        ],
        out_specs=pl.BlockSpec((1, c2, h * w), lambda i: (i, 0, 0)),
        compiler_params=par,
    )(x1.reshape(b, c1, h * w), w1, b_1x1.reshape(c2, 1).astype(jnp.float32))

    uy = _interp_matrix(h)
    uxt = _interp_matrix(w).T
    uyb = jnp.broadcast_to(uy[None], (c2, h2, h))
    y_up = pl.pallas_call(
        _make_upsample_kernel(c2, h, w),
        out_shape=jax.ShapeDtypeStruct((b, c2, h2, w2), jnp.bfloat16),
        grid=(b,),
        in_specs=[
            pl.BlockSpec((1, c2 * h, w), lambda i: (i, 0, 0)),
            pl.BlockSpec((c2, h2, h), lambda i: (0, 0, 0)),
            pl.BlockSpec((w, w2), lambda i: (0, 0)),
        ],
        out_specs=pl.BlockSpec((1, c2, h2, w2), lambda i: (i, 0, 0, 0)),
        compiler_params=par,
    )(y1.reshape(b, c2 * h, w), uyb, uxt)

    w_a = jnp.transpose(w_conv_a, (2, 3, 1, 0)).astype(jnp.bfloat16)
    scale_a, shift_a = _fold_bn(b_conv_a, bn_a_gamma, bn_a_beta,
                                bn_a_mean, bn_a_var)
    za = pl.pallas_call(
        _make_conv3x3_kernel(2, h2, w2, forms=("cm", "cm"),
                             nchw_out=False),
        out_shape=jax.ShapeDtypeStruct((b, n, c_out), jnp.bfloat16),
        grid=(b,),
        in_specs=[
            pl.BlockSpec((1, c2, n), lambda i: (i, 0, 0)),
            pl.BlockSpec((1, c2, n), lambda i: (i, 0, 0)),
            pl.BlockSpec((3, 3, c2, c_out), lambda i: (0, 0, 0, 0)),
            pl.BlockSpec((3, 3, c2, c_out), lambda i: (0, 0, 0, 0)),
            pl.BlockSpec((1, c_out), lambda i: (0, 0)),
            pl.BlockSpec((1, c_out), lambda i: (0, 0)),
        ],
        out_specs=pl.BlockSpec((1, n, c_out), lambda i: (i, 0, 0)),
        compiler_params=par,
        scratch_shapes=[pltpu.VMEM((n + 2 * guard, c2), jnp.bfloat16)
                        for _ in range(2)],
    )(x2.reshape(b, c2, n), y_up.reshape(b, c2, n),
      w_a[:, :, :c2, :], w_a[:, :, c2:, :], scale_a, shift_a)

    w_b = jnp.transpose(w_conv_b, (2, 3, 1, 0)).astype(jnp.bfloat16)
    scale_b, shift_b = _fold_bn(b_conv_b, bn_b_gamma, bn_b_beta,
                                bn_b_mean, bn_b_var)
    zb = pl.pallas_call(
        _make_conv3x3_kernel(1, h2, w2, forms=("rows3",), nchw_out=True),
        out_shape=jax.ShapeDtypeStruct((b, c_out, n), jnp.float32),
        grid=(b,),
        in_specs=[
            pl.BlockSpec((1, n, c_out), lambda i: (i, 0, 0)),
            pl.BlockSpec((3, 3, c_out, c_out), lambda i: (0, 0, 0, 0)),
            pl.BlockSpec((1, c_out), lambda i: (0, 0)),
            pl.BlockSpec((1, c_out), lambda i: (0, 0)),
        ],
        out_specs=pl.BlockSpec((1, c_out, n), lambda i: (i, 0, 0)),
        compiler_params=par,
        scratch_shapes=[pltpu.VMEM((n + 2 * guard, c_out), jnp.bfloat16)],
    )(za, w_b, scale_b, shift_b)

    return zb.reshape(b, c_out, h2, w2)
